# baseline (speedup 1.0000x reference)
import dataclasses
import functools

import jax
import jax.numpy as jnp
from jax import lax
from jax.experimental import pallas as pl
from jax.experimental.pallas import tpu as pltpu
from jax.experimental.pallas import tpu_sc as plsc

_DIMS = [119, 4, 12, 12, 10, 6, 6, 2, 2, 7, 1, 49, 61, 2, 5, 7]
_EMB = 128
_NW = 32
_CH = 128

_GROUPS = [(0, 10), (12, 7), (11, 8), (2, 3), (4, 13), (9, 15, 1), (5, 6, 14)]
_JDIMS = []
for _g in _GROUPS:
    _p = 1
    for _f in _g:
        _p *= _DIMS[_f]
    _JDIMS.append(_p)
_POFF = [0]
for _d in _JDIMS[:-1]:
    _POFF.append(_POFF[-1] + _d)
_JTOT = _POFF[-1] + _JDIMS[-1]


def _vsplat(vec, i):
    idx = jnp.full((16, 1), i, jnp.int32)
    return lax.gather(
        vec,
        idx,
        lax.GatherDimensionNumbers(
            offset_dims=(), collapsed_slice_dims=(0,), start_index_map=(0,)
        ),
        slice_sizes=(1,),
        mode=lax.GatherScatterMode.PROMISE_IN_BOUNDS,
    )


def _prep_body(*refs):
    ws, out_ref = refs[:-1], refs[-1]
    for g, off, dj in zip(_GROUPS, _POFF, _JDIMS):
        blk = ws[g[0]][:, :]
        for f in g[1:]:
            wf = ws[f][:, :]
            blk = (blk[:, None, :] + wf[None, :, :]).reshape(-1, _EMB)
        out_ref[off : off + dj, :] = blk


def kernel(x, W0, W1, W2, W3, W4, W5, W6, W7, W8, W9, W10, W11, W12, W13, W14, W15):
    Ws = [W0, W1, W2, W3, W4, W5, W6, W7, W8, W9, W10, W11, W12, W13, W14, W15]
    n = x.shape[0]
    n_feat = x.shape[1]
    tab = pl.pallas_call(
        _prep_body,
        out_shape=jax.ShapeDtypeStruct((_JTOT, _EMB), jnp.float32),
    )(*Ws)
    jb = tab.astype(jnp.bfloat16)
    ptab = jax.lax.bitcast_convert_type(
        jnp.stack([jb[:, : _EMB // 2], jb[:, _EMB // 2 :]], axis=-1), jnp.int32
    ).reshape(_JTOT * (_EMB // 2))

    n_full = n // _CH
    rem = n % _CH
    last_full = n_full - 1
    n_in = (n_full + (1 if rem else 0)) * _CH
    x_t = jnp.pad(x, ((0, n_in - n), (0, 0))).T
    rounds = (-(-n_full // _NW) + 1) // 2

    mesh = plsc.VectorSubcoreMesh(core_axis_name="c", subcore_axis_name="s")
    cparams = pltpu.CompilerParams()
    if "needs_layout_passes" in pltpu.CompilerParams.__dataclass_fields__:
        cparams = dataclasses.replace(cparams, needs_layout_passes=False)

    @functools.partial(
        pl.kernel,
        compiler_params=cparams,
        out_type=jax.ShapeDtypeStruct((n, _EMB), jnp.float32),
        mesh=mesh,
        scratch_types=[
            pltpu.VMEM((_JTOT * (_EMB // 2),), jnp.int32),
            pltpu.VMEM((n_feat, _CH), jnp.int32),
            pltpu.VMEM((n_feat, _CH), jnp.int32),
            pltpu.VMEM((_CH, _EMB), jnp.float32),
            pltpu.VMEM((_CH, _EMB), jnp.float32),
            pltpu.SemaphoreType.DMA,
            pltpu.SemaphoreType.DMA,
            pltpu.SemaphoreType.DMA,
            pltpu.SemaphoreType.DMA,
        ],
    )
    def _emb_sum(
        xt_hbm, tab_hbm, out_hbm, tab_v, idx_a, idx_b, out_a, out_b,
        sem_ia, sem_ib, sem_oa, sem_ob,
    ):
        wid = lax.axis_index("s") * 2 + lax.axis_index("c")
        pltpu.sync_copy(tab_hbm, tab_v)
        lanes = lax.iota(jnp.int32, 16)
        tabs = [
            tab_v.at[pl.ds(16 * j, _JTOT * (_EMB // 2) - 16 * j)]
            for j in range(_EMB // 32)
        ]

        def issue_idx(cid, idx_v, sem):
            pltpu.async_copy(xt_hbm.at[:, pl.ds(cid * _CH, _CH)], idx_v, sem)

        def wait_in(idx_v, sem):
            pltpu.make_async_copy(xt_hbm.at[:, pl.ds(0, _CH)], idx_v, sem).wait()

        def wait_out(out_v, sem):
            pltpu.make_async_copy(out_hbm.at[pl.ds(0, _CH), :], out_v, sem).wait()

        def compute(idx_v, out_v):
            @pl.loop(0, _CH // 16)
            def _blk(bi):
                b16 = bi * 16
                vs = []
                for g, off in zip(_GROUPS, _POFF):
                    v = idx_v[g[0], pl.ds(b16, 16)]
                    for f in g[1:]:
                        v = v * _DIMS[f] + idx_v[f, pl.ds(b16, 16)]
                    vs.append((v + off) * (_EMB // 2))

                @plsc.parallel_loop(0, 16, unroll=4)
                def _row(l):
                    acc = [None] * (_EMB // 32)
                    for i in range(len(_GROUPS)):
                        sl = _vsplat(vs[i], l) + lanes
                        for j in range(_EMB // 32):
                            g = plsc.load_gather(tabs[j], [sl])
                            gb = plsc.bitcast(g, jnp.bfloat16)
                            acc[j] = gb if acc[j] is None else acc[j] + gb
                    row = b16 + l
                    for j in range(_EMB // 32):
                        av, bv = plsc.unpack(acc[j], format=plsc.PackFormat.INTERLEAVED)
                        out_v[row, pl.ds(16 * j, 16)] = av
                        out_v[row, pl.ds(_EMB // 2 + 16 * j, 16)] = bv

        @pl.when(wid <= last_full)
        def _prime():
            issue_idx(wid, idx_a, sem_ia)

        @pl.loop(0, rounds)
        def _round(r):
            cid0 = wid + _NW * 2 * r
            cid1 = cid0 + _NW
            cid0n = cid0 + 2 * _NW

            @pl.when(cid1 <= last_full)
            def _():
                issue_idx(cid1, idx_b, sem_ib)

            @pl.when(cid0 <= last_full)
            def _half_a():
                @pl.when(r > 0)
                def _():
                    wait_out(out_a, sem_oa)

                wait_in(idx_a, sem_ia)
                compute(idx_a, out_a)
                pltpu.async_copy(out_a, out_hbm.at[pl.ds(cid0 * _CH, _CH), :], sem_oa)

            @pl.when(cid0n <= last_full)
            def _():
                issue_idx(cid0n, idx_a, sem_ia)

            @pl.when(cid1 <= last_full)
            def _half_b():
                @pl.when(r > 0)
                def _():
                    wait_out(out_b, sem_ob)

                wait_in(idx_b, sem_ib)
                compute(idx_b, out_b)
                pltpu.async_copy(out_b, out_hbm.at[pl.ds(cid1 * _CH, _CH), :], sem_ob)

        wait_out(out_a, sem_oa)
        wait_out(out_b, sem_ob)

        if rem:
            @pl.when(wid == _NW - 1)
            def _tail():
                pltpu.sync_copy(xt_hbm.at[:, pl.ds(n_full * _CH, _CH)], idx_a)
                compute(idx_a, out_a)
                pltpu.sync_copy(
                    out_a.at[pl.ds(0, rem), :],
                    out_hbm.at[pl.ds(n_full * _CH, rem), :],
                )

    return _emb_sum(x_t, ptab)


# speedup vs baseline: 1.8912x; 1.8912x over previous
import dataclasses
import functools

import jax
import jax.numpy as jnp
from jax import lax
from jax.experimental import pallas as pl
from jax.experimental.pallas import tpu as pltpu
from jax.experimental.pallas import tpu_sc as plsc

_DIMS = [119, 4, 12, 12, 10, 6, 6, 2, 2, 7, 1, 49, 61, 2, 5, 7]
_EMB = 128
_NW = 32
_CH = 128

_GROUPS = [(0, 10), (12, 7), (11, 8), (2, 3), (4, 13), (9, 15, 1), (5, 6, 14)]
_JDIMS = []
for _g in _GROUPS:
    _p = 1
    for _f in _g:
        _p *= _DIMS[_f]
    _JDIMS.append(_p)
_POFF = [0]
for _d in _JDIMS[:-1]:
    _POFF.append(_POFF[-1] + _d)
_JTOT = _POFF[-1] + _JDIMS[-1]


def _vsplat(vec, i):
    idx = jnp.full((16, 1), i, jnp.int32)
    return lax.gather(
        vec,
        idx,
        lax.GatherDimensionNumbers(
            offset_dims=(), collapsed_slice_dims=(0,), start_index_map=(0,)
        ),
        slice_sizes=(1,),
        mode=lax.GatherScatterMode.PROMISE_IN_BOUNDS,
    )


def _prep_body(*refs):
    ws, out_ref = refs[:-1], refs[-1]
    for g, off, dj in zip(_GROUPS, _POFF, _JDIMS):
        blk = ws[g[0]][:, :]
        for f in g[1:]:
            wf = ws[f][:, :]
            blk = (blk[:, None, :] + wf[None, :, :]).reshape(-1, _EMB)
        out_ref[off : off + dj, :] = blk


def kernel(x, W0, W1, W2, W3, W4, W5, W6, W7, W8, W9, W10, W11, W12, W13, W14, W15):
    Ws = [W0, W1, W2, W3, W4, W5, W6, W7, W8, W9, W10, W11, W12, W13, W14, W15]
    n = x.shape[0]
    n_feat = x.shape[1]
    tab = pl.pallas_call(
        _prep_body,
        out_shape=jax.ShapeDtypeStruct((_JTOT, _EMB), jnp.float32),
    )(*Ws)
    jb = tab.astype(jnp.bfloat16)
    ptab = jax.lax.bitcast_convert_type(
        jnp.stack([jb[:, : _EMB // 2], jb[:, _EMB // 2 :]], axis=-1), jnp.int32
    ).reshape(_JTOT * (_EMB // 2))

    n_full = n // _CH
    rem = n % _CH
    last_full = n_full - 1
    n_in = (n_full + (1 if rem else 0)) * _CH
    x_t = jnp.pad(x, ((0, n_in - n), (0, 0))).T
    rounds = (-(-n_full // _NW) + 1) // 2

    mesh = plsc.VectorSubcoreMesh(core_axis_name="c", subcore_axis_name="s")
    cparams = pltpu.CompilerParams()
    if "needs_layout_passes" in pltpu.CompilerParams.__dataclass_fields__:
        cparams = dataclasses.replace(cparams, needs_layout_passes=False)

    @functools.partial(
        pl.kernel,
        compiler_params=cparams,
        out_type=jax.ShapeDtypeStruct((n, _EMB), jnp.float32),
        mesh=mesh,
        scratch_types=[
            pltpu.VMEM((_JTOT * (_EMB // 2),), jnp.int32),
            pltpu.VMEM((n_feat, _CH), jnp.int32),
            pltpu.VMEM((n_feat, _CH), jnp.int32),
            pltpu.VMEM((_CH, _EMB), jnp.float32),
            pltpu.VMEM((_CH, _EMB), jnp.float32),
            pltpu.SemaphoreType.DMA,
            pltpu.SemaphoreType.DMA,
            pltpu.SemaphoreType.DMA,
            pltpu.SemaphoreType.DMA,
        ],
    )
    def _emb_sum(
        xt_hbm, tab_hbm, out_hbm, tab_v, idx_a, idx_b, out_a, out_b,
        sem_ia, sem_ib, sem_oa, sem_ob,
    ):
        wid = lax.axis_index("s") * 2 + lax.axis_index("c")
        pltpu.sync_copy(tab_hbm, tab_v)
        lanes = lax.iota(jnp.int32, 16)
        tabs = [
            tab_v.at[pl.ds(16 * j, _JTOT * (_EMB // 2) - 16 * j)]
            for j in range(_EMB // 32)
        ]

        def issue_idx(cid, idx_v, sem):
            pltpu.async_copy(xt_hbm.at[:, pl.ds(cid * _CH, _CH)], idx_v, sem)

        def wait_in(idx_v, sem):
            pltpu.make_async_copy(xt_hbm.at[:, pl.ds(0, _CH)], idx_v, sem).wait()

        def wait_out(out_v, sem):
            pltpu.make_async_copy(out_hbm.at[pl.ds(0, _CH), :], out_v, sem).wait()

        def compute(idx_v, out_v):
            @plsc.parallel_loop(0, _CH // 16)
            def _blk(bi):
                b16 = bi * 16
                vs = []
                for g, off in zip(_GROUPS, _POFF):
                    v = idx_v[g[0], pl.ds(b16, 16)]
                    for f in g[1:]:
                        v = v * _DIMS[f] + idx_v[f, pl.ds(b16, 16)]
                    vs.append((v + off) * (_EMB // 2))

                @plsc.parallel_loop(0, 16, unroll=2)
                def _row(l):
                    acc = [None] * (_EMB // 32)
                    for i in range(len(_GROUPS)):
                        sl = _vsplat(vs[i], l) + lanes
                        for j in range(_EMB // 32):
                            g = plsc.load_gather(tabs[j], [sl])
                            gb = plsc.bitcast(g, jnp.bfloat16)
                            acc[j] = gb if acc[j] is None else acc[j] + gb
                    row = b16 + l
                    for j in range(_EMB // 32):
                        av, bv = plsc.unpack(acc[j], format=plsc.PackFormat.INTERLEAVED)
                        out_v[row, pl.ds(16 * j, 16)] = av
                        out_v[row, pl.ds(_EMB // 2 + 16 * j, 16)] = bv

        @pl.when(wid <= last_full)
        def _prime():
            issue_idx(wid, idx_a, sem_ia)

        @pl.loop(0, rounds)
        def _round(r):
            cid0 = wid + _NW * 2 * r
            cid1 = cid0 + _NW
            cid0n = cid0 + 2 * _NW

            @pl.when(cid1 <= last_full)
            def _():
                issue_idx(cid1, idx_b, sem_ib)

            @pl.when(cid0 <= last_full)
            def _half_a():
                @pl.when(r > 0)
                def _():
                    wait_out(out_a, sem_oa)

                wait_in(idx_a, sem_ia)
                compute(idx_a, out_a)
                pltpu.async_copy(out_a, out_hbm.at[pl.ds(cid0 * _CH, _CH), :], sem_oa)

            @pl.when(cid0n <= last_full)
            def _():
                issue_idx(cid0n, idx_a, sem_ia)

            @pl.when(cid1 <= last_full)
            def _half_b():
                @pl.when(r > 0)
                def _():
                    wait_out(out_b, sem_ob)

                wait_in(idx_b, sem_ib)
                compute(idx_b, out_b)
                pltpu.async_copy(out_b, out_hbm.at[pl.ds(cid1 * _CH, _CH), :], sem_ob)

        wait_out(out_a, sem_oa)
        wait_out(out_b, sem_ob)

        if rem:
            @pl.when(wid == _NW - 1)
            def _tail():
                pltpu.sync_copy(xt_hbm.at[:, pl.ds(n_full * _CH, _CH)], idx_a)
                compute(idx_a, out_a)
                pltpu.sync_copy(
                    out_a.at[pl.ds(0, rem), :],
                    out_hbm.at[pl.ds(n_full * _CH, rem), :],
                )

    return _emb_sum(x_t, ptab)
